# baseline (speedup 1.0000x reference)
import jax
import jax.numpy as jnp
from jax import lax
from jax.experimental import pallas as pl
from jax.experimental.pallas import tpu as pltpu
from jax.experimental.pallas import tpu_sc as plsc

_N_BINS = 10
_NCOL = 21

_SC_CORES = 2
_SC_SUBCORES = 16
_SC_LANES = 16
_SC_WORKERS = _SC_CORES * _SC_SUBCORES
_TBL_PAD = 224

_BR = 4000
_GRP = 5
_SUB = _BR // _GRP
_HALF = _SUB // 2
_SC_CHUNK = _BR
_SLANE = _GRP * _NCOL


def _sc_body(pack_hbm, labels_hbm, match_hbm, out_hbm,
             rows_v, lab_v, m_v, table_v, fold_v):
    wid = lax.axis_index("s") * _SC_CORES + lax.axis_index("c")
    nchunks = labels_hbm.shape[0] // _SC_CHUNK

    zz = jnp.zeros((16,), jnp.float32)
    for g in range(_SC_LANES * _TBL_PAD // 16):
        table_v[pl.ds(g * 16, 16)] = zz

    lane = lax.iota(jnp.int32, 16)

    n_outer = (nchunks + _SC_WORKERS - 1) // _SC_WORKERS
    for t in range(n_outer):
        chunk = wid + t * _SC_WORKERS

        @pl.when(chunk < nchunks)
        def _do():
            base = chunk * _SC_CHUNK
            pltpu.sync_copy(pack_hbm.at[pl.ds(chunk * _HALF, _HALF)], rows_v)
            pltpu.sync_copy(labels_hbm.at[pl.ds(base, _SC_CHUNK)], lab_v)
            pltpu.sync_copy(match_hbm.at[pl.ds(base, _SC_CHUNK)], m_v)

            for k in range(_GRP):
                for u in range(2):
                    def step(g, carry, k=k, u=u):
                        off = k * _SUB + u * _HALF + g * 16
                        lab16 = lab_v[pl.ds(off, 16)]
                        m16 = m_v[pl.ds(off, 16)]
                        row16 = lane + g * 16
                        col16 = k * _NCOL + lab16
                        w16 = plsc.load_gather(rows_v, [row16, col16])
                        s16 = jnp.bitwise_and(
                            lax.shift_right_logical(w16, 16 * u), 0xFFFF)
                        valid = s16 >= 1
                        buck = jnp.where(valid, (s16 - 1) * _NCOL + lab16, 0)
                        val = jnp.where(valid, m16, 0.0)
                        plsc.addupdate_scatter(
                            table_v, [lane * _TBL_PAD + buck], val)
                        return carry

                    lax.fori_loop(0, _HALF // 16, step, 0)

    for g in range(_TBL_PAD // 16):
        acc = table_v[pl.ds(g * 16, 16)]
        for l in range(1, _SC_LANES):
            acc = acc + table_v[pl.ds(l * _TBL_PAD + g * 16, 16)]
        fold_v[pl.ds(g * 16, 16)] = acc

    pltpu.sync_copy(fold_v, out_hbm.at[wid])


def _sc_tp(pack, labels, match_f):
    mesh = plsc.VectorSubcoreMesh(core_axis_name="c", subcore_axis_name="s")
    fn = pl.kernel(
        _sc_body,
        out_type=jax.ShapeDtypeStruct((_SC_WORKERS, _TBL_PAD), jnp.float32),
        mesh=mesh,
        scratch_types=[
            pltpu.VMEM((_HALF, 128), jnp.int32),
            pltpu.VMEM((_SC_CHUNK,), jnp.int32),
            pltpu.VMEM((_SC_CHUNK,), jnp.float32),
            pltpu.VMEM((_SC_LANES * _TBL_PAD,), jnp.float32),
            pltpu.VMEM((_TBL_PAD,), jnp.float32),
        ],
        compiler_params=pltpu.CompilerParams(needs_layout_passes=False),
    )
    return fn(pack, labels, match_f)


def _dense_body(edges_ref, pba_ref, pbb_ref, pack_ref, acc_ref):
    i = pl.program_id(0)

    @pl.when(i == 0)
    def _init():
        acc_ref[...] = jnp.zeros_like(acc_ref)

    pb21 = jnp.concatenate([pba_ref[...], pbb_ref[...]], axis=0)
    pb = jnp.concatenate(
        [pb21[k * _SUB:(k + 1) * _SUB, :] for k in range(_GRP)], axis=1)
    edges = edges_ref[...]
    ones = jnp.ones((1, _SUB), jnp.float32)
    dn = (((1,), (0,)), ((), ()))
    rows = []
    masks = []
    for j in range(_N_BINS):
        e = edges[0:1, j:j + 1]
        m = (pb > e).astype(jnp.float32)
        masks.append(m)
        rows.append(lax.dot_general(ones, m, dn,
                                    preferred_element_type=jnp.float32))
    for j in range(_N_BINS):
        e = edges[0:1, j:j + 1]
        pm = jnp.where(pb > e, pb, 0.0)
        rows.append(lax.dot_general(ones, pm, dn,
                                    preferred_element_type=jnp.float32))
    acc_ref[0:2 * _N_BINS, 0:_SLANE] += jnp.concatenate(rows, axis=0)

    s_f = masks[0]
    for j in range(1, _N_BINS):
        s_f = s_f + masks[j]
    s32 = s_f.astype(jnp.int32)
    word = jnp.bitwise_or(s32[0:_HALF, :],
                          lax.shift_left(s32[_HALF:_SUB, :], 16))

    a = acc_ref[...]
    folded = (a[:, 0:21] + a[:, 21:42] + a[:, 42:63] + a[:, 63:84]
              + a[:, 84:105])
    filler = jnp.concatenate(
        [lax.bitcast_convert_type(folded, jnp.int32),
         jnp.zeros((_HALF - 32, _NCOL), jnp.int32)], axis=0)
    pack_ref[0] = jnp.concatenate(
        [word, filler, jnp.zeros((_HALF, 2), jnp.int32)], axis=1)


def _combine_body(stats_ref, tp_ref, out_ref):
    folded = lax.bitcast_convert_type(
        stats_ref[...][:, _SLANE:_SLANE + _NCOL], jnp.float32)
    tp3 = tp_ref[...]
    tpb = jnp.sum(tp3, axis=0)
    cnt = folded[0:_N_BINS, :]
    sp = folded[_N_BINS:2 * _N_BINS, :]
    z = jnp.zeros((1, _NCOL), jnp.float32)
    ns = cnt - jnp.concatenate([cnt[1:, :], z], axis=0)
    spb = sp - jnp.concatenate([sp[1:, :], z], axis=0)
    total = jnp.sum(ns, axis=0, keepdims=True)
    mp = spb / jnp.maximum(ns, 1.0)
    pr = tpb / jnp.maximum(ns, 1e-12)
    pbw = ns / jnp.maximum(total, 1.0)
    term = jnp.where(ns > 0.0, pbw * jnp.square(mp - pr), 0.0)
    s_c = jnp.sum(term, axis=0, keepdims=True)
    sq = jnp.square(jnp.sqrt(s_c))
    lidx = lax.broadcasted_iota(jnp.int32, (1, _NCOL), 1)
    sq = jnp.where(lidx < _NCOL - 1, sq, 0.0)
    out_ref[...] = jnp.sqrt(jnp.sum(sq, axis=1, keepdims=True) / (_NCOL - 1))


def kernel(probas, labels, matchings):
    n, ncol = probas.shape
    edges_full = jnp.linspace(0.0, 1.0, _N_BINS + 1, dtype=jnp.float32)
    edges16 = jnp.zeros((1, 16), jnp.float32).at[0, :11].set(edges_full)
    match_f = matchings.astype(jnp.float32)

    pack = pl.pallas_call(
        _dense_body,
        grid=(n // _BR,),
        in_specs=[
            pl.BlockSpec((1, 16), lambda i: (0, 0)),
            pl.BlockSpec((_BR // 2, _NCOL), lambda i: (2 * i, 0)),
            pl.BlockSpec((_BR // 2, _NCOL), lambda i: (2 * i + 1, 0)),
        ],
        out_specs=pl.BlockSpec((1, _HALF, 128), lambda i: (i, 0, 0)),
        out_shape=jax.ShapeDtypeStruct((n // _BR, _HALF, 128), jnp.int32),
        scratch_shapes=[pltpu.VMEM((32, 128), jnp.float32)],
        compiler_params=pltpu.CompilerParams(
            dimension_semantics=("arbitrary",)),
    )(edges16, probas, probas)
    pack = pack.reshape(n // _BR * _HALF, 128)

    tp_part = _sc_tp(pack, labels, match_f)
    tp3 = tp_part[:, :_N_BINS * _NCOL].reshape(_SC_WORKERS, _N_BINS, _NCOL)

    nblk = n // _BR
    out = pl.pallas_call(
        _combine_body,
        grid=(1,),
        in_specs=[
            pl.BlockSpec((32, 128), lambda i: ((nblk - 1) * _HALF // 32, 0)),
            pl.BlockSpec((_SC_WORKERS, _N_BINS, _NCOL), lambda i: (0, 0, 0)),
        ],
        out_specs=pl.BlockSpec((1, 1), lambda i: (0, 0)),
        out_shape=jax.ShapeDtypeStruct((1, 1), jnp.float32),
    )(pack, tp3)
    return out[0, 0]


# speedup vs baseline: 1.0845x; 1.0845x over previous
import jax
import jax.numpy as jnp
from jax import lax
from jax.experimental import pallas as pl
from jax.experimental.pallas import tpu as pltpu
from jax.experimental.pallas import tpu_sc as plsc

_N_BINS = 10
_NCOL = 21

_SC_CORES = 2
_SC_SUBCORES = 16
_SC_LANES = 16
_SC_WORKERS = _SC_CORES * _SC_SUBCORES
_SC_CHUNK = 400
_TBL_PAD = 224

_BR = 4000
_GRP = 5
_SUB = _BR // _GRP
_SLANE = _GRP * _NCOL


def _sc_body(edges_hbm, probas_hbm, labels_hbm, match_hbm, out_hbm,
             rows_v, lab_v, m_v, edges_v, table_v, fold_v):
    wid = lax.axis_index("s") * _SC_CORES + lax.axis_index("c")
    nchunks = probas_hbm.shape[0] // _SC_CHUNK

    zz = jnp.zeros((16,), jnp.float32)
    for g in range(_SC_LANES * _TBL_PAD // 16):
        table_v[pl.ds(g * 16, 16)] = zz

    pltpu.sync_copy(edges_hbm, edges_v)
    ev = [edges_v[j, :] for j in range(_N_BINS)]
    lane = lax.iota(jnp.int32, 16)

    n_outer = (nchunks + _SC_WORKERS - 1) // _SC_WORKERS
    for t in range(n_outer):
        chunk = wid + t * _SC_WORKERS

        @pl.when(chunk < nchunks)
        def _do():
            base = chunk * _SC_CHUNK
            pltpu.sync_copy(probas_hbm.at[pl.ds(base, _SC_CHUNK)], rows_v)
            pltpu.sync_copy(labels_hbm.at[pl.ds(base, _SC_CHUNK)], lab_v)
            pltpu.sync_copy(match_hbm.at[pl.ds(base, _SC_CHUNK)], m_v)

            def step(g, carry):
                off = g * 16
                lab16 = lab_v[pl.ds(off, 16)]
                m16 = m_v[pl.ds(off, 16)]
                row16 = lane + off
                q16 = plsc.load_gather(rows_v, [row16, lab16])
                s = jnp.zeros((16,), jnp.int32)
                for j in range(_N_BINS):
                    s = s + jnp.where(q16 > ev[j], 1, 0)
                valid = s >= 1
                buck = jnp.where(valid, (s - 1) * _NCOL + lab16, 0)
                val = jnp.where(valid, m16, 0.0)
                plsc.addupdate_scatter(table_v, [lane * _TBL_PAD + buck], val)
                return carry

            lax.fori_loop(0, _SC_CHUNK // 16, step, 0)

    for g in range(_TBL_PAD // 16):
        acc = table_v[pl.ds(g * 16, 16)]
        for l in range(1, _SC_LANES):
            acc = acc + table_v[pl.ds(l * _TBL_PAD + g * 16, 16)]
        fold_v[pl.ds(g * 16, 16)] = acc

    pltpu.sync_copy(fold_v, out_hbm.at[wid])


def _sc_tp(probas, labels, match_f, edges_b):
    mesh = plsc.VectorSubcoreMesh(core_axis_name="c", subcore_axis_name="s")
    fn = pl.kernel(
        _sc_body,
        out_type=jax.ShapeDtypeStruct((_SC_WORKERS, _TBL_PAD), jnp.float32),
        mesh=mesh,
        scratch_types=[
            pltpu.VMEM((_SC_CHUNK, _NCOL), jnp.float32),
            pltpu.VMEM((_SC_CHUNK,), jnp.int32),
            pltpu.VMEM((_SC_CHUNK,), jnp.float32),
            pltpu.VMEM((_N_BINS, 16), jnp.float32),
            pltpu.VMEM((_SC_LANES * _TBL_PAD,), jnp.float32),
            pltpu.VMEM((_TBL_PAD,), jnp.float32),
        ],
        compiler_params=pltpu.CompilerParams(needs_layout_passes=False),
    )
    return fn(edges_b, probas, labels, match_f)


def _dense_body(edges_ref, pb_ref, out_ref, acc_ref):
    i = pl.program_id(0)

    @pl.when(i == 0)
    def _init():
        acc_ref[...] = jnp.zeros_like(acc_ref)

    pb21 = pb_ref[...]
    pb = jnp.concatenate(
        [pb21[k * _SUB:(k + 1) * _SUB, :] for k in range(_GRP)], axis=1)
    edges = edges_ref[...]
    ones = jnp.ones((1, _SUB), jnp.float32)
    dn = (((1,), (0,)), ((), ()))
    rows = []
    for j in range(_N_BINS):
        e = edges[0:1, j:j + 1]
        m = (pb > e).astype(jnp.float32)
        rows.append(lax.dot_general(ones, m, dn,
                                    preferred_element_type=jnp.float32))
    for j in range(_N_BINS):
        e = edges[0:1, j:j + 1]
        pm = jnp.where(pb > e, pb, 0.0)
        rows.append(lax.dot_general(ones, pm, dn,
                                    preferred_element_type=jnp.float32))
    acc_ref[0:2 * _N_BINS, 0:_SLANE] += jnp.concatenate(rows, axis=0)
    out_ref[0] = acc_ref[...]


def _combine_body(stats_ref, tp_ref, out_ref):
    a = stats_ref[0]
    folded = (a[:, 0:21] + a[:, 21:42] + a[:, 42:63] + a[:, 63:84]
              + a[:, 84:105])
    tp3 = tp_ref[...]
    tpb = jnp.sum(tp3, axis=0)
    cnt = folded[0:_N_BINS, :]
    sp = folded[_N_BINS:2 * _N_BINS, :]
    z = jnp.zeros((1, _NCOL), jnp.float32)
    ns = cnt - jnp.concatenate([cnt[1:, :], z], axis=0)
    spb = sp - jnp.concatenate([sp[1:, :], z], axis=0)
    total = jnp.sum(ns, axis=0, keepdims=True)
    mp = spb / jnp.maximum(ns, 1.0)
    pr = tpb / jnp.maximum(ns, 1e-12)
    pbw = ns / jnp.maximum(total, 1.0)
    term = jnp.where(ns > 0.0, pbw * jnp.square(mp - pr), 0.0)
    s_c = jnp.sum(term, axis=0, keepdims=True)
    sq = jnp.square(jnp.sqrt(s_c))
    lidx = lax.broadcasted_iota(jnp.int32, (1, _NCOL), 1)
    sq = jnp.where(lidx < _NCOL - 1, sq, 0.0)
    out_ref[...] = jnp.sqrt(jnp.sum(sq, axis=1, keepdims=True) / (_NCOL - 1))


def kernel(probas, labels, matchings):
    n, ncol = probas.shape
    edges_full = jnp.linspace(0.0, 1.0, _N_BINS + 1, dtype=jnp.float32)
    edges16 = jnp.zeros((1, 16), jnp.float32).at[0, :11].set(edges_full)
    edges_b = jnp.broadcast_to(edges_full[:_N_BINS, None], (_N_BINS, 16))
    match_f = matchings.astype(jnp.float32)

    nblk = n // _BR
    stats = pl.pallas_call(
        _dense_body,
        grid=(nblk,),
        in_specs=[
            pl.BlockSpec((1, 16), lambda i: (0, 0)),
            pl.BlockSpec((_BR, _NCOL), lambda i: (i, 0)),
        ],
        out_specs=pl.BlockSpec((1, 32, 128), lambda i: (i, 0, 0)),
        out_shape=jax.ShapeDtypeStruct((nblk, 32, 128), jnp.float32),
        scratch_shapes=[pltpu.VMEM((32, 128), jnp.float32)],
        compiler_params=pltpu.CompilerParams(
            dimension_semantics=("arbitrary",)),
    )(edges16, probas)

    tp_part = _sc_tp(probas, labels, match_f, edges_b)
    tp3 = tp_part[:, :_N_BINS * _NCOL].reshape(_SC_WORKERS, _N_BINS, _NCOL)

    out = pl.pallas_call(
        _combine_body,
        grid=(1,),
        in_specs=[
            pl.BlockSpec((1, 32, 128), lambda i: (nblk - 1, 0, 0)),
            pl.BlockSpec((_SC_WORKERS, _N_BINS, _NCOL), lambda i: (0, 0, 0)),
        ],
        out_specs=pl.BlockSpec((1, 1), lambda i: (0, 0)),
        out_shape=jax.ShapeDtypeStruct((1, 1), jnp.float32),
    )(stats, tp3)
    return out[0, 0]


# speedup vs baseline: 1.0892x; 1.0043x over previous
import jax
import jax.numpy as jnp
from jax import lax
from jax.experimental import pallas as pl
from jax.experimental.pallas import tpu as pltpu
from jax.experimental.pallas import tpu_sc as plsc

_N_BINS = 10
_NCOL = 21

_SC_CORES = 2
_SC_SUBCORES = 16
_SC_LANES = 16
_SC_WORKERS = _SC_CORES * _SC_SUBCORES
_SC_CHUNK = 400
_TBL_PAD = 224

_BR = 20000
_GRP = 5
_SUB = _BR // _GRP
_SLANE = _GRP * _NCOL


def _sc_body(edges_hbm, probas_hbm, labels_hbm, match_hbm, out_hbm,
             rows_v, lab_v, m_v, edges_v, table_v, fold_v):
    wid = lax.axis_index("s") * _SC_CORES + lax.axis_index("c")
    nchunks = probas_hbm.shape[0] // _SC_CHUNK

    zz = jnp.zeros((16,), jnp.float32)
    for g in range(_SC_LANES * _TBL_PAD // 16):
        table_v[pl.ds(g * 16, 16)] = zz

    pltpu.sync_copy(edges_hbm, edges_v)
    ev = [edges_v[j, :] for j in range(_N_BINS)]
    lane = lax.iota(jnp.int32, 16)

    n_outer = (nchunks + _SC_WORKERS - 1) // _SC_WORKERS
    for t in range(n_outer):
        chunk = wid + t * _SC_WORKERS

        @pl.when(chunk < nchunks)
        def _do():
            base = chunk * _SC_CHUNK
            pltpu.sync_copy(probas_hbm.at[pl.ds(base, _SC_CHUNK)], rows_v)
            pltpu.sync_copy(labels_hbm.at[pl.ds(base, _SC_CHUNK)], lab_v)
            pltpu.sync_copy(match_hbm.at[pl.ds(base, _SC_CHUNK)], m_v)

            def step(g, carry):
                off = g * 16
                lab16 = lab_v[pl.ds(off, 16)]
                m16 = m_v[pl.ds(off, 16)]
                row16 = lane + off
                q16 = plsc.load_gather(rows_v, [row16, lab16])
                s = jnp.zeros((16,), jnp.int32)
                for j in range(_N_BINS):
                    s = s + jnp.where(q16 > ev[j], 1, 0)
                valid = s >= 1
                buck = jnp.where(valid, (s - 1) * _NCOL + lab16, 0)
                val = jnp.where(valid, m16, 0.0)
                plsc.addupdate_scatter(table_v, [lane * _TBL_PAD + buck], val)
                return carry

            lax.fori_loop(0, _SC_CHUNK // 16, step, 0)

    for g in range(_TBL_PAD // 16):
        acc = table_v[pl.ds(g * 16, 16)]
        for l in range(1, _SC_LANES):
            acc = acc + table_v[pl.ds(l * _TBL_PAD + g * 16, 16)]
        fold_v[pl.ds(g * 16, 16)] = acc

    pltpu.sync_copy(fold_v, out_hbm.at[wid])


def _sc_tp(probas, labels, match_f, edges_b):
    mesh = plsc.VectorSubcoreMesh(core_axis_name="c", subcore_axis_name="s")
    fn = pl.kernel(
        _sc_body,
        out_type=jax.ShapeDtypeStruct((_SC_WORKERS, _TBL_PAD), jnp.float32),
        mesh=mesh,
        scratch_types=[
            pltpu.VMEM((_SC_CHUNK, _NCOL), jnp.float32),
            pltpu.VMEM((_SC_CHUNK,), jnp.int32),
            pltpu.VMEM((_SC_CHUNK,), jnp.float32),
            pltpu.VMEM((_N_BINS, 16), jnp.float32),
            pltpu.VMEM((_SC_LANES * _TBL_PAD,), jnp.float32),
            pltpu.VMEM((_TBL_PAD,), jnp.float32),
        ],
        compiler_params=pltpu.CompilerParams(needs_layout_passes=False),
    )
    return fn(edges_b, probas, labels, match_f)


def _dense_body(edges_ref, pb_ref, out_ref, acc_ref):
    i = pl.program_id(0)

    @pl.when(i == 0)
    def _init():
        acc_ref[...] = jnp.zeros_like(acc_ref)

    pb21 = pb_ref[...]
    pb = jnp.concatenate(
        [pb21[k * _SUB:(k + 1) * _SUB, :] for k in range(_GRP)], axis=1)
    edges = edges_ref[...]
    ones = jnp.ones((1, _SUB), jnp.float32)
    dn = (((1,), (0,)), ((), ()))
    rows = []
    for j in range(_N_BINS):
        e = edges[0:1, j:j + 1]
        m = (pb > e).astype(jnp.float32)
        rows.append(lax.dot_general(ones, m, dn,
                                    preferred_element_type=jnp.float32))
    for j in range(_N_BINS):
        e = edges[0:1, j:j + 1]
        pm = jnp.where(pb > e, pb, 0.0)
        rows.append(lax.dot_general(ones, pm, dn,
                                    preferred_element_type=jnp.float32))
    acc_ref[0:2 * _N_BINS, 0:_SLANE] += jnp.concatenate(rows, axis=0)
    out_ref[0] = acc_ref[...]


def _combine_body(stats_ref, tp_ref, out_ref):
    a = stats_ref[0]
    folded = (a[:, 0:21] + a[:, 21:42] + a[:, 42:63] + a[:, 63:84]
              + a[:, 84:105])
    tp3 = tp_ref[...]
    tpb = jnp.sum(tp3, axis=0)
    cnt = folded[0:_N_BINS, :]
    sp = folded[_N_BINS:2 * _N_BINS, :]
    z = jnp.zeros((1, _NCOL), jnp.float32)
    ns = cnt - jnp.concatenate([cnt[1:, :], z], axis=0)
    spb = sp - jnp.concatenate([sp[1:, :], z], axis=0)
    total = jnp.sum(ns, axis=0, keepdims=True)
    mp = spb / jnp.maximum(ns, 1.0)
    pr = tpb / jnp.maximum(ns, 1e-12)
    pbw = ns / jnp.maximum(total, 1.0)
    term = jnp.where(ns > 0.0, pbw * jnp.square(mp - pr), 0.0)
    s_c = jnp.sum(term, axis=0, keepdims=True)
    sq = jnp.square(jnp.sqrt(s_c))
    lidx = lax.broadcasted_iota(jnp.int32, (1, _NCOL), 1)
    sq = jnp.where(lidx < _NCOL - 1, sq, 0.0)
    out_ref[...] = jnp.sqrt(jnp.sum(sq, axis=1, keepdims=True) / (_NCOL - 1))


def kernel(probas, labels, matchings):
    n, ncol = probas.shape
    edges_full = jnp.linspace(0.0, 1.0, _N_BINS + 1, dtype=jnp.float32)
    edges16 = jnp.zeros((1, 16), jnp.float32).at[0, :11].set(edges_full)
    edges_b = jnp.broadcast_to(edges_full[:_N_BINS, None], (_N_BINS, 16))
    match_f = matchings.astype(jnp.float32)

    nblk = n // _BR
    stats = pl.pallas_call(
        _dense_body,
        grid=(nblk,),
        in_specs=[
            pl.BlockSpec((1, 16), lambda i: (0, 0)),
            pl.BlockSpec((_BR, _NCOL), lambda i: (i, 0)),
        ],
        out_specs=pl.BlockSpec((1, 32, 128), lambda i: (i, 0, 0)),
        out_shape=jax.ShapeDtypeStruct((nblk, 32, 128), jnp.float32),
        scratch_shapes=[pltpu.VMEM((32, 128), jnp.float32)],
        compiler_params=pltpu.CompilerParams(
            dimension_semantics=("arbitrary",)),
    )(edges16, probas)

    tp_part = _sc_tp(probas, labels, match_f, edges_b)
    tp3 = tp_part[:, :_N_BINS * _NCOL].reshape(_SC_WORKERS, _N_BINS, _NCOL)

    out = pl.pallas_call(
        _combine_body,
        grid=(1,),
        in_specs=[
            pl.BlockSpec((1, 32, 128), lambda i: (nblk - 1, 0, 0)),
            pl.BlockSpec((_SC_WORKERS, _N_BINS, _NCOL), lambda i: (0, 0, 0)),
        ],
        out_specs=pl.BlockSpec((1, 1), lambda i: (0, 0)),
        out_shape=jax.ShapeDtypeStruct((1, 1), jnp.float32),
    )(stats, tp3)
    return out[0, 0]
